# baseline (speedup 1.0000x reference)
import jax
import jax.numpy as jnp
from jax import lax
from jax.experimental import pallas as pl
from jax.experimental.pallas import tpu as pltpu
from jax.experimental.pallas import tpu_sc as plsc

_NC = 2
_NS = 16
_NW = _NC * _NS
_L = 16
_D = 32
_V = 1000000
_SUB = 8
_LANES = 128
_TCOLS = -(-_V // _LANES)
_SEG = _TCOLS * _SUB * _LANES


def _mf_body(uidx_hbm, iidx_hbm, uflat_hbm, iflat_hbm, out_hbm,
             uidx_v, iidx_v, urT_v, irT_v, out_v,
             sem_u, sem_i):
    b_per_w = uidx_v.shape[0]
    wid = lax.axis_index("s") * _NC + lax.axis_index("c")
    base = wid * b_per_w

    pltpu.sync_copy(uidx_hbm.at[pl.ds(base, b_per_w)], uidx_v)
    pltpu.sync_copy(iidx_hbm.at[pl.ds(base, b_per_w)], iidx_v)

    copies = []
    for j in range(_D):
        copies.append(pltpu.async_copy(
            uflat_hbm.at[pl.ds(j * _V, _V)].at[uidx_v],
            urT_v.at[pl.ds(j * b_per_w, b_per_w)], sem_u))
        copies.append(pltpu.async_copy(
            iflat_hbm.at[pl.ds(j * _V, _V)].at[iidx_v],
            irT_v.at[pl.ds(j * b_per_w, b_per_w)], sem_i))
    for cp in copies:
        cp.wait()

    def chunk_body(c, carry):
        b0 = c * _L
        acc = jnp.zeros((_L,), jnp.float32)
        for j in range(_D):
            u = urT_v[pl.ds(j * b_per_w + b0, _L)]
            i = irT_v[pl.ds(j * b_per_w + b0, _L)]
            acc = acc + u * i
        out_v[pl.ds(b0, _L)] = 1.0 / (1.0 + jnp.exp(-acc))
        return carry

    lax.fori_loop(0, b_per_w // _L, chunk_body, 0)
    pltpu.sync_copy(out_v, out_hbm.at[pl.ds(base, b_per_w)])


def _flat_physical(table):
    return table.T.reshape(-1)


def kernel(user_indices, item_indices, user_table, item_table):
    B = user_indices.shape[0]
    assert B % (_NW * _L) == 0
    assert user_table.shape == (_V, _D)
    b_per_w = B // _NW
    mesh = plsc.VectorSubcoreMesh(core_axis_name="c", subcore_axis_name="s",
                                  num_cores=_NC, num_subcores=_NS)
    run = pl.kernel(
        _mf_body,
        out_type=jax.ShapeDtypeStruct((B,), jnp.float32),
        mesh=mesh,
        compiler_params=pltpu.CompilerParams(needs_layout_passes=False,
                                             use_tc_tiling_on_sc=False),
        scratch_types=[
            pltpu.VMEM((b_per_w,), jnp.int32),
            pltpu.VMEM((b_per_w,), jnp.int32),
            pltpu.VMEM((_D * b_per_w,), jnp.float32),
            pltpu.VMEM((_D * b_per_w,), jnp.float32),
            pltpu.VMEM((b_per_w,), jnp.float32),
            pltpu.SemaphoreType.DMA,
            pltpu.SemaphoreType.DMA,
        ],
    )
    return run(user_indices, item_indices,
               _flat_physical(user_table), _flat_physical(item_table))
